# baseline (speedup 1.0000x reference)
import functools

import jax
import jax.numpy as jnp
from jax import lax
from jax.experimental import pallas as pl
from jax.experimental.pallas import tpu as pltpu
from jax.experimental.pallas import tpu_sc as plsc

B = 16384
D = 64
DP = 128
NC = 2
NS = 16
NW = NC * NS
BPW = B // NW
CH = 128
NCHUNK = BPW // CH


def _body(xu_h, xv_h, uid_t, iid_t, out, xu_v, xv_v, rows_v, sem):
    wid = lax.axis_index("s") * NC + lax.axis_index("c")
    base = wid * BPW
    pltpu.sync_copy(xu_h.at[pl.ds(base, BPW)], xu_v)
    pltpu.sync_copy(xv_h.at[pl.ds(base, BPW)], xv_v)
    for t in range(2):
        idx_v = (xu_v, xv_v)[t]
        tab = (uid_t, iid_t)[t]
        for j in range(NCHUNK):
            pltpu.async_copy(
                tab.at[idx_v.at[pl.ds(j * CH, CH)]], rows_v, sem).wait()
            pltpu.sync_copy(
                rows_v,
                out.at[pl.ds(base + j * CH, CH), pl.ds(t * DP, DP)])


@jax.jit
def _lookup(xu, xv, uid_t, iid_t):
    mesh = plsc.VectorSubcoreMesh(core_axis_name="c", subcore_axis_name="s")
    f = functools.partial(
        pl.kernel,
        mesh=mesh,
        out_type=jax.ShapeDtypeStruct((B, 2 * DP), jnp.float32),
        scratch_types=[
            pltpu.VMEM((BPW,), jnp.int32),
            pltpu.VMEM((BPW,), jnp.int32),
            pltpu.VMEM((CH, DP), jnp.float32),
            pltpu.SemaphoreType.DMA,
        ],
    )(_body)
    return f(xu, xv, uid_t, iid_t)


def kernel(x, uid_table, iid_table):
    xi = x.astype(jnp.int32)
    up = jnp.pad(uid_table, ((0, 0), (0, DP - D)))
    ip = jnp.pad(iid_table[:1000000], ((0, 0), (0, DP - D)))
    o2 = _lookup(xi[:, 0], xi[:, 1], up, ip)
    return o2.reshape(B, 2, DP)[:, :, :D]


# speedup vs baseline: 1.9778x; 1.9778x over previous
import functools

import jax
import jax.numpy as jnp
from jax import lax
from jax.experimental import pallas as pl
from jax.experimental.pallas import tpu as pltpu
from jax.experimental.pallas import tpu_sc as plsc

B = 16384
D = 64
NC = 2
NS = 16
NW = NC * NS
BPW = B // NW
CH = 16
NCHUNK = BPW // CH
TROWS = 8


def _scalar(vec, lane_iota, e):
    return jnp.max(jnp.where(lane_iota == e, vec, jnp.int32(-1)))


def _body(xu_h, xv_h, uid_tab, iid_tab, out,
          xu_v, xv_v, tiles_u, tiles_i, rows_c, sem):
    wid = lax.axis_index("s") * NC + lax.axis_index("c")
    base = wid * BPW
    pltpu.sync_copy(xu_h.at[pl.ds(base, BPW)], xu_v)
    pltpu.sync_copy(xv_h.at[pl.ds(base, BPW)], xv_v)
    lane_iota = lax.iota(jnp.int32, 16)

    def chunk(c, _):
        vec_u = xu_v[pl.ds(c * CH, CH)]
        vec_i = xv_v[pl.ds(c * CH, CH)]
        rus = [_scalar(vec_u, lane_iota, e) for e in range(CH)]
        ris = [_scalar(vec_i, lane_iota, e) for e in range(CH)]
        copies = []
        for e in range(CH):
            copies.append(pltpu.async_copy(
                uid_tab.at[pl.ds(rus[e] >> 3, 1)], tiles_u.at[pl.ds(e, 1)], sem))
            copies.append(pltpu.async_copy(
                iid_tab.at[pl.ds(ris[e] >> 3, 1)], tiles_i.at[pl.ds(e, 1)], sem))
        for cp in copies:
            cp.wait()
        for e in range(CH):
            su = rus[e] & 7
            si = ris[e] & 7
            for q in range(D // 16):
                rows_c[e, pl.ds(16 * q, 16)] = tiles_u[e, su, pl.ds(16 * q, 16)]
                rows_c[e, pl.ds(D + 16 * q, 16)] = tiles_i[e, si, pl.ds(16 * q, 16)]
        pltpu.sync_copy(rows_c, out.at[pl.ds(base + c * CH, CH)])
        return ()

    lax.fori_loop(0, NCHUNK, chunk, (), unroll=False)


@jax.jit
def _lookup(xu, xv, uid_t, iid_t):
    mesh = plsc.VectorSubcoreMesh(core_axis_name="c", subcore_axis_name="s")
    f = functools.partial(
        pl.kernel,
        mesh=mesh,
        out_type=jax.ShapeDtypeStruct((B, 2 * D), jnp.float32),
        scratch_types=[
            pltpu.VMEM((BPW,), jnp.int32),
            pltpu.VMEM((BPW,), jnp.int32),
            pltpu.VMEM((CH, TROWS, D), jnp.float32),
            pltpu.VMEM((CH, TROWS, D), jnp.float32),
            pltpu.VMEM((CH, 2 * D), jnp.float32),
            pltpu.SemaphoreType.DMA,
        ],
        compiler_params=pltpu.CompilerParams(needs_layout_passes=False),
    )(_body)
    return f(xu, xv, uid_t, iid_t)


def kernel(x, uid_table, iid_table):
    xi = x.astype(jnp.int32)
    ut = uid_table.reshape(125000, TROWS, D)
    it = iid_table[:1000000].reshape(125000, TROWS, D)
    o2 = _lookup(xi[:, 0], xi[:, 1], ut, it)
    return o2.reshape(B, 2, D)
